# baseline (speedup 1.0000x reference)
import jax
import jax.numpy as jnp
from jax import lax
from jax.experimental import pallas as pl
from jax.experimental.pallas import tpu as pltpu
from jax.experimental.pallas import tpu_sc as plsc

N = 10000
D = 256
HD = 128
NP = 10112
NS = 16
RPT = NP // NS
E = 160000
CH = 80
NCH = 125
ET = NCH * CH
BN = 632


def _mm_body(x_ref, w_ref, va_ref, vd_ref, hc_ref, as_ref, ad_ref):
    hf = pl.program_id(1)
    h = jnp.dot(x_ref[...], w_ref[...], preferred_element_type=jnp.float32)
    hc_ref[...] = h
    pa = jnp.dot(h, va_ref[...], preferred_element_type=jnp.float32)
    pd = jnp.dot(h, vd_ref[...], preferred_element_type=jnp.float32)

    @pl.when(hf == 0)
    def _():
        as_ref[...] = pa
        ad_ref[...] = pd

    @pl.when(hf == 1)
    def _():
        as_ref[...] = as_ref[...] + pa
        ad_ref[...] = ad_ref[...] + pd


_G = NP // BN

_mm = pl.pallas_call(
    _mm_body,
    grid=(_G, 2),
    in_specs=[
        pl.BlockSpec((BN, D), lambda i, hf: (i, 0)),
        pl.BlockSpec((D, HD), lambda i, hf: (0, hf)),
        pl.BlockSpec((HD, 1), lambda i, hf: (hf, 0)),
        pl.BlockSpec((HD, 1), lambda i, hf: (hf, 0)),
    ],
    out_specs=[
        pl.BlockSpec((BN, HD), lambda i, hf: (hf * _G + i, 0)),
        pl.BlockSpec((BN, 1), lambda i, hf: (i, 0)),
        pl.BlockSpec((BN, 1), lambda i, hf: (i, 0)),
    ],
    out_shape=[
        jax.ShapeDtypeStruct((2 * NP, HD), jnp.float32),
        jax.ShapeDtypeStruct((NP, 1), jnp.float32),
        jax.ShapeDtypeStruct((NP, 1), jnp.float32),
    ],
)


def _edge_body(src_ref, dst_ref, as_ref, ad_ref, hcat_ref,
               z2_ref, z1_ref, out0_ref, out1_ref, s_ref,
               av, si, di, sio, adg, wv, rows,
               isem, agsem, gsem, ssem, osem, out_sh, s_sh):
    cid = lax.axis_index("c")
    sid = lax.axis_index("s")
    hoff = cid * NP
    ebase = sid * ET

    pltpu.sync_copy(z2_ref.at[pl.ds(sid * RPT, RPT)],
                    out_sh.at[pl.ds(sid * RPT, RPT)])

    @pl.when(sid == 0)
    def _():
        pltpu.sync_copy(z1_ref, s_sh)

    pltpu.sync_copy(as_ref, av)
    plsc.subcore_barrier()

    def start_idx(k, m6):
        base = ebase + k * CH
        pltpu.async_copy(src_ref.at[pl.ds(base, CH)], si.at[m6], isem.at[m6])
        pltpu.async_copy(dst_ref.at[pl.ds(base, CH)], di.at[m6], isem.at[m6])

    def wait_idx(m6):
        pltpu.make_async_copy(src_ref.at[pl.ds(0, CH)], si.at[m6],
                              isem.at[m6]).wait()
        pltpu.make_async_copy(dst_ref.at[pl.ds(0, CH)], di.at[m6],
                              isem.at[m6]).wait()

    def start_gathers(m6, b3):
        for j in range(CH // 16):
            sio[m6, pl.ds(j * 16, 16)] = si[m6, pl.ds(j * 16, 16)] + hoff
        pltpu.async_copy(ad_ref.at[di.at[m6]], adg.at[b3], agsem.at[b3])
        pltpu.async_copy(hcat_ref.at[sio.at[m6]], rows.at[b3], gsem.at[b3])

    def wait_gathers(m6, b3):
        pltpu.make_async_copy(ad_ref.at[di.at[m6]], adg.at[b3],
                              agsem.at[b3]).wait()
        pltpu.make_async_copy(hcat_ref.at[sio.at[m6]], rows.at[b3],
                              gsem.at[b3]).wait()

    def wait_oscatter(b3, m6):
        pltpu.make_async_copy(rows.at[b3], out_sh.at[di.at[m6]],
                              osem.at[b3]).wait()

    def wait_sscatter(b3, m6):
        pltpu.make_async_copy(wv.at[b3], s_sh.at[di.at[m6]],
                              ssem.at[b3]).wait()

    def stage(k, m6, do_drain=True, do_idx=True, do_pref=True):
        b3 = m6 % 3
        n6 = (m6 + 1) % 6
        n3 = (m6 + 1) % 3
        p6 = (m6 + 2) % 6
        if do_drain:
            wait_oscatter(n3, n6)
            wait_sscatter(n3, n6)
        if do_idx:
            start_idx(k + 2, p6)
        if do_pref:
            wait_idx(n6)
            start_gathers(n6, n3)
        wait_gathers(m6, b3)
        for j in range(CH // 16):
            sv = si[m6, pl.ds(j * 16, 16)]
            e = plsc.load_gather(av, [sv]) + adg[b3, pl.ds(j * 16, 16)]
            e = jnp.where(e > 0, e, 0.2 * e)
            wv[b3, pl.ds(j * 16, 16)] = jnp.exp(e)
        pltpu.async_copy(wv.at[b3], s_sh.at[di.at[m6]], ssem.at[b3], add=True)

        @plsc.parallel_loop(0, CH, 1, unroll=4)
        def _(r):
            wb = plsc.load_gather(wv, [jnp.zeros((16,), jnp.int32) + b3,
                                       jnp.zeros((16,), jnp.int32) + r])
            for f in range(HD // 16):
                rows[b3, r, pl.ds(f * 16, 16)] = (
                    rows[b3, r, pl.ds(f * 16, 16)] * wb)

        pltpu.async_copy(rows.at[b3], out_sh.at[di.at[m6]], osem.at[b3],
                         add=True)

    start_idx(0, 0)
    start_idx(1, 1)
    wait_idx(0)
    start_gathers(0, 0)
    stage(0, 0, do_drain=False)
    stage(1, 1, do_drain=False)

    def body(jj, _):
        k0 = 2 + jj * 6
        for t in range(6):
            stage(k0 + t, (2 + t) % 6)
        return 0

    lax.fori_loop(0, (NCH - 5) // 6, body, 0)
    stage(NCH - 3, (NCH - 3) % 6)
    stage(NCH - 2, (NCH - 2) % 6, do_idx=False)
    stage(NCH - 1, (NCH - 1) % 6, do_idx=False, do_pref=False)
    wait_oscatter((NCH - 2) % 3, (NCH - 2) % 6)
    wait_sscatter((NCH - 2) % 3, (NCH - 2) % 6)
    wait_oscatter((NCH - 1) % 3, (NCH - 1) % 6)
    wait_sscatter((NCH - 1) % 3, (NCH - 1) % 6)
    plsc.subcore_barrier()

    @pl.when(cid == 0)
    def _():
        pltpu.sync_copy(out_sh.at[pl.ds(sid * RPT, RPT)],
                        out0_ref.at[pl.ds(sid * RPT, RPT)])

    @pl.when(cid == 1)
    def _():
        pltpu.sync_copy(out_sh.at[pl.ds(sid * RPT, RPT)],
                        out1_ref.at[pl.ds(sid * RPT, RPT)])

    @pl.when(jnp.logical_and(cid == 0, sid == 0))
    def _():
        pltpu.sync_copy(s_sh, s_ref)


_edge = pl.kernel(
    _edge_body,
    out_type=[
        jax.ShapeDtypeStruct((NP, HD), jnp.float32),
        jax.ShapeDtypeStruct((NP, HD), jnp.float32),
        jax.ShapeDtypeStruct((NP,), jnp.float32),
    ],
    mesh=plsc.VectorSubcoreMesh(core_axis_name="c", subcore_axis_name="s"),
    compiler_params=pltpu.CompilerParams(needs_layout_passes=False),
    scratch_types=[
        pltpu.VMEM((NP,), jnp.float32),
        pltpu.VMEM((6, CH), jnp.int32),
        pltpu.VMEM((6, CH), jnp.int32),
        pltpu.VMEM((6, CH), jnp.int32),
        pltpu.VMEM((3, CH), jnp.float32),
        pltpu.VMEM((3, CH), jnp.float32),
        pltpu.VMEM((3, CH, HD), jnp.float32),
        pltpu.SemaphoreType.DMA((6,)),
        pltpu.SemaphoreType.DMA((3,)),
        pltpu.SemaphoreType.DMA((3,)),
        pltpu.SemaphoreType.DMA((3,)),
        pltpu.SemaphoreType.DMA((3,)),
        pltpu.VMEM_SHARED((NP, HD), jnp.float32),
        pltpu.VMEM_SHARED((NP,), jnp.float32),
    ],
)


def _ln_math(x, o0, o1, h0, h1, s, asv, adv, b, g, be):
    es = asv + adv
    ws = jnp.exp(jnp.where(es > 0, es, 0.2 * es))
    inv = 1.0 / (s + ws + 1e-16)
    att = jnp.concatenate(
        [(o0 + ws * h0) * inv, (o1 + ws * h1) * inv], axis=1)
    t = x + att + b
    mu = jnp.mean(t, axis=1, keepdims=True)
    var = jnp.mean((t - mu) ** 2, axis=1, keepdims=True)
    return (t - mu) * lax.rsqrt(var + 1e-5) * g + be


def _ln_body(x_ref, o0_ref, o1_ref, h0_ref, h1_ref, s_ref, as_ref, ad_ref,
             b_ref, g_ref, be_ref, y_ref):
    y_ref[...] = _ln_math(
        x_ref[...], o0_ref[...], o1_ref[...], h0_ref[...], h1_ref[...],
        s_ref[...], as_ref[...], ad_ref[...], b_ref[...], g_ref[...],
        be_ref[...])


_ln = pl.pallas_call(
    _ln_body,
    grid=(_G,),
    in_specs=[
        pl.BlockSpec((BN, D), lambda i: (i, 0)),
        pl.BlockSpec((BN, HD), lambda i: (i, 0)),
        pl.BlockSpec((BN, HD), lambda i: (i, 0)),
        pl.BlockSpec((BN, HD), lambda i: (i, 0)),
        pl.BlockSpec((BN, HD), lambda i: (_G + i, 0)),
        pl.BlockSpec((BN, 1), lambda i: (i, 0)),
        pl.BlockSpec((BN, 1), lambda i: (i, 0)),
        pl.BlockSpec((BN, 1), lambda i: (i, 0)),
        pl.BlockSpec((1, D), lambda i: (0, 0)),
        pl.BlockSpec((1, D), lambda i: (0, 0)),
        pl.BlockSpec((1, D), lambda i: (0, 0)),
    ],
    out_specs=pl.BlockSpec((BN, D), lambda i: (i, 0)),
    out_shape=jax.ShapeDtypeStruct((NP, D), jnp.float32),
)


def _lnmm_body(x_ref, o0_ref, o1_ref, h0_ref, h1_ref, s_ref, as_ref, ad_ref,
               b_ref, g_ref, be_ref, wn_ref, van_ref, vdn_ref,
               y_ref, hcn_ref, nas_ref, nad_ref):
    hf = pl.program_id(1)
    y = _ln_math(
        x_ref[...], o0_ref[...], o1_ref[...], h0_ref[...], h1_ref[...],
        s_ref[...], as_ref[...], ad_ref[...], b_ref[...], g_ref[...],
        be_ref[...])

    @pl.when(hf == 0)
    def _():
        y_ref[...] = y

    hn = jnp.dot(y, wn_ref[...], preferred_element_type=jnp.float32)
    hcn_ref[...] = hn
    pa = jnp.dot(hn, van_ref[...], preferred_element_type=jnp.float32)
    pd = jnp.dot(hn, vdn_ref[...], preferred_element_type=jnp.float32)

    @pl.when(hf == 0)
    def _():
        nas_ref[...] = pa
        nad_ref[...] = pd

    @pl.when(hf == 1)
    def _():
        nas_ref[...] = nas_ref[...] + pa
        nad_ref[...] = nad_ref[...] + pd


_lnmm = pl.pallas_call(
    _lnmm_body,
    grid=(_G, 2),
    in_specs=[
        pl.BlockSpec((BN, D), lambda i, hf: (i, 0)),
        pl.BlockSpec((BN, HD), lambda i, hf: (i, 0)),
        pl.BlockSpec((BN, HD), lambda i, hf: (i, 0)),
        pl.BlockSpec((BN, HD), lambda i, hf: (i, 0)),
        pl.BlockSpec((BN, HD), lambda i, hf: (_G + i, 0)),
        pl.BlockSpec((BN, 1), lambda i, hf: (i, 0)),
        pl.BlockSpec((BN, 1), lambda i, hf: (i, 0)),
        pl.BlockSpec((BN, 1), lambda i, hf: (i, 0)),
        pl.BlockSpec((1, D), lambda i, hf: (0, 0)),
        pl.BlockSpec((1, D), lambda i, hf: (0, 0)),
        pl.BlockSpec((1, D), lambda i, hf: (0, 0)),
        pl.BlockSpec((D, HD), lambda i, hf: (0, hf)),
        pl.BlockSpec((HD, 1), lambda i, hf: (hf, 0)),
        pl.BlockSpec((HD, 1), lambda i, hf: (hf, 0)),
    ],
    out_specs=[
        pl.BlockSpec((BN, D), lambda i, hf: (i, 0)),
        pl.BlockSpec((BN, HD), lambda i, hf: (hf * _G + i, 0)),
        pl.BlockSpec((BN, 1), lambda i, hf: (i, 0)),
        pl.BlockSpec((BN, 1), lambda i, hf: (i, 0)),
    ],
    out_shape=[
        jax.ShapeDtypeStruct((NP, D), jnp.float32),
        jax.ShapeDtypeStruct((2 * NP, HD), jnp.float32),
        jax.ShapeDtypeStruct((NP, 1), jnp.float32),
        jax.ShapeDtypeStruct((NP, 1), jnp.float32),
    ],
)


@jax.jit
def _run(x, edge_index, W0, a_src0, a_dst0, b0, g0, be0,
         W1, a_src1, a_dst1, b1, g1, be1):
    xp = jnp.zeros((NP, D), jnp.float32).at[:N].set(x)
    src = edge_index[0].astype(jnp.int32)
    dst = edge_index[1].astype(jnp.int32)
    z2 = jnp.zeros((NP, HD), jnp.float32)
    z1 = jnp.zeros((NP,), jnp.float32)

    hcat, asrc, adst = _mm(xp, W0, a_src0.reshape(D, 1), a_dst0.reshape(D, 1))
    out0, out1, s = _edge(src, dst, asrc.reshape(NP), adst.reshape(NP),
                          hcat, z2, z1)
    y1, hcat2, asrc2, adst2 = _lnmm(
        xp, out0, out1, hcat, hcat, s.reshape(NP, 1), asrc, adst,
        b0.reshape(1, D), g0.reshape(1, D), be0.reshape(1, D),
        W1, a_src1.reshape(D, 1), a_dst1.reshape(D, 1))
    out0b, out1b, sb = _edge(src, dst, asrc2.reshape(NP), adst2.reshape(NP),
                             hcat2, z2, z1)
    y2 = _ln(y1, out0b, out1b, hcat2, hcat2, sb.reshape(NP, 1), asrc2, adst2,
             b1.reshape(1, D), g1.reshape(1, D), be1.reshape(1, D))
    return y2[:N]


def kernel(x, edge_index, W0, a_src0, a_dst0, b0, g0, be0,
           W1, a_src1, a_dst1, b1, g1, be1):
    return _run(x, edge_index, W0, a_src0, a_dst0, b0, g0, be0,
                W1, a_src1, a_dst1, b1, g1, be1)
